# baseline (device time: 69745 ns/iter reference)
import jax
import jax.numpy as jnp
from jax import lax
from jax.experimental import pallas as pl
from jax.experimental.pallas import tpu as pltpu

M = 1024
N = 1024
H = M // 2


def kernel(dy, W):
    def body(dy_ref, w_ref, out_ref, ybuf,
             ysend_sem, yrecv_sem, xsend_sem, xrecv_sem):
        my_x = lax.axis_index("x")
        my_y = lax.axis_index("y")

        barrier_sem = pltpu.get_barrier_semaphore()
        pl.semaphore_signal(
            barrier_sem, inc=1,
            device_id=(my_x, 1 - my_y), device_id_type=pl.DeviceIdType.MESH)
        pl.semaphore_signal(
            barrier_sem, inc=1,
            device_id=(1 - my_x, my_y), device_id_type=pl.DeviceIdType.MESH)
        pl.semaphore_wait(barrier_sem, 2)

        row0 = my_x * H

        a = dy_ref[pl.ds(row0, H), :]
        p = lax.dot_general(
            a, w_ref[...],
            dimension_numbers=(((1,), (1,)), ((), ())),
            preferred_element_type=jnp.float32,
        )
        out_ref[pl.ds(row0, H), :] = p

        y_rdma = pltpu.make_async_remote_copy(
            src_ref=out_ref.at[pl.ds(row0, H)],
            dst_ref=ybuf,
            send_sem=ysend_sem,
            recv_sem=yrecv_sem,
            device_id=(my_x, 1 - my_y),
            device_id_type=pl.DeviceIdType.MESH,
        )
        y_rdma.start()
        y_rdma.wait()

        out_ref[pl.ds(row0, H), :] = out_ref[pl.ds(row0, H), :] + ybuf[...]

        x_rdma = pltpu.make_async_remote_copy(
            src_ref=out_ref.at[pl.ds(row0, H)],
            dst_ref=out_ref.at[pl.ds(row0, H)],
            send_sem=xsend_sem,
            recv_sem=xrecv_sem,
            device_id=(1 - my_x, my_y),
            device_id_type=pl.DeviceIdType.MESH,
        )
        x_rdma.start()
        x_rdma.wait()

    return pl.pallas_call(
        body,
        out_shape=jax.ShapeDtypeStruct((M, N), jnp.float32),
        in_specs=[
            pl.BlockSpec(memory_space=pltpu.VMEM),
            pl.BlockSpec(memory_space=pltpu.VMEM),
        ],
        out_specs=pl.BlockSpec(memory_space=pltpu.VMEM),
        scratch_shapes=[
            pltpu.VMEM((H, N), jnp.float32),
            pltpu.SemaphoreType.DMA,
            pltpu.SemaphoreType.DMA,
            pltpu.SemaphoreType.DMA,
            pltpu.SemaphoreType.DMA,
        ],
        compiler_params=pltpu.CompilerParams(collective_id=0),
    )(dy, W)


# device time: 63244 ns/iter; 1.1028x vs baseline; 1.1028x over previous
import jax
import jax.numpy as jnp
from jax import lax
from jax.experimental import pallas as pl
from jax.experimental.pallas import tpu as pltpu

M = 1024
N = 1024
H = M // 2
K = 8
CH = H // K


def kernel(dy, W):
    def body(dy_ref, w_ref, out_ref, ybuf,
             ysend_sems, yrecv_sems, xsend_sems, xrecv_sems):
        my_x = lax.axis_index("x")
        my_y = lax.axis_index("y")

        barrier_sem = pltpu.get_barrier_semaphore()
        pl.semaphore_signal(
            barrier_sem, inc=1,
            device_id=(my_x, 1 - my_y), device_id_type=pl.DeviceIdType.MESH)
        pl.semaphore_signal(
            barrier_sem, inc=1,
            device_id=(1 - my_x, my_y), device_id_type=pl.DeviceIdType.MESH)
        pl.semaphore_wait(barrier_sem, 2)

        row0 = my_x * H

        def chunk_slice(c):
            return pl.ds(row0 + c * CH, CH)

        def y_copy(c):
            return pltpu.make_async_remote_copy(
                src_ref=out_ref.at[chunk_slice(c)],
                dst_ref=ybuf.at[c],
                send_sem=ysend_sems.at[c],
                recv_sem=yrecv_sems.at[c],
                device_id=(my_x, 1 - my_y),
                device_id_type=pl.DeviceIdType.MESH,
            )

        def x_copy(c):
            return pltpu.make_async_remote_copy(
                src_ref=out_ref.at[chunk_slice(c)],
                dst_ref=out_ref.at[chunk_slice(c)],
                send_sem=xsend_sems.at[c],
                recv_sem=xrecv_sems.at[c],
                device_id=(1 - my_x, my_y),
                device_id_type=pl.DeviceIdType.MESH,
            )

        for c in range(K):
            a = dy_ref[chunk_slice(c), :]
            p = lax.dot_general(
                a, w_ref[...],
                dimension_numbers=(((1,), (1,)), ((), ())),
                preferred_element_type=jnp.float32,
            )
            out_ref[chunk_slice(c), :] = p
            y_copy(c).start()

        for c in range(K):
            yc = y_copy(c)
            yc.wait_send()
            yc.wait_recv()
            out_ref[chunk_slice(c), :] = out_ref[chunk_slice(c), :] + ybuf[c]
            x_copy(c).start()

        for c in range(K):
            x_copy(c).wait()

    return pl.pallas_call(
        body,
        out_shape=jax.ShapeDtypeStruct((M, N), jnp.float32),
        in_specs=[
            pl.BlockSpec(memory_space=pltpu.VMEM),
            pl.BlockSpec(memory_space=pltpu.VMEM),
        ],
        out_specs=pl.BlockSpec(memory_space=pltpu.VMEM),
        scratch_shapes=[
            pltpu.VMEM((K, CH, N), jnp.float32),
            pltpu.SemaphoreType.DMA((K,)),
            pltpu.SemaphoreType.DMA((K,)),
            pltpu.SemaphoreType.DMA((K,)),
            pltpu.SemaphoreType.DMA((K,)),
        ],
        compiler_params=pltpu.CompilerParams(collective_id=0),
    )(dy, W)
